# baseline (speedup 1.0000x reference)
import functools

import jax
import jax.numpy as jnp
from jax import lax
from jax.experimental import pallas as pl
from jax.experimental.pallas import tpu as pltpu
from jax.experimental.pallas import tpu_sc as plsc

TEMP = 0.1
BASE_TEMP = 1.0
FLAT = 120000
KT = 3840
NSTEPS = (FLAT + KT - 1) // KT
NEG = -1e30
NIDX = 384
_INTERPRET = False


def _sc_gather(labf, idx_pad):
    info = plsc.get_sparse_core_info()
    nc = info.num_cores
    mesh = plsc.VectorSubcoreMesh(core_axis_name="c", subcore_axis_name="s")

    @functools.partial(
        pl.kernel, mesh=mesh,
        out_type=jax.ShapeDtypeStruct((512,), jnp.float32),
        scratch_types=[
            pltpu.VMEM((16,), jnp.int32),
            pltpu.VMEM((16,), jnp.float32),
            pltpu.SemaphoreType.DMA,
        ],
    )
    def gather_kernel(lab_hbm, idx_hbm, out_hbm, idxv, outv, sem):
        wid = lax.axis_index("s") * nc + lax.axis_index("c")
        base = wid * 16
        pltpu.sync_copy(idx_hbm.at[pl.ds(base, 16)], idxv)
        pltpu.async_copy(lab_hbm.at[idxv], outv, sem).wait()
        pltpu.sync_copy(outv, out_hbm.at[pl.ds(base, 16)])

    return gather_kernel(labf, idx_pad)


def _mm_kernel(xc_ref, xf_ref, w1_ref, oc_ref, of_ref):
    k = pl.program_id(0)

    @pl.when(k == 0)
    def _init():
        oc_ref[...] = jnp.zeros_like(oc_ref)
        of_ref[...] = jnp.zeros_like(of_ref)

    @pl.when(k < NSTEPS - 1)
    def _full():
        w1 = w1_ref[...]
        oc_ref[...] += jnp.dot(xc_ref[...], w1, preferred_element_type=jnp.float32)
        of_ref[...] += jnp.dot(xf_ref[...], w1, preferred_element_type=jnp.float32)

    @pl.when(k == NSTEPS - 1)
    def _last():
        rem = FLAT - (NSTEPS - 1) * KT
        colmask = lax.broadcasted_iota(jnp.int32, (1, KT), 1) < rem
        rowmask = lax.broadcasted_iota(jnp.int32, (KT, 1), 0) < rem
        xc = jnp.where(colmask, xc_ref[...], 0.0)
        xf = jnp.where(colmask, xf_ref[...], 0.0)
        w1 = jnp.where(rowmask, w1_ref[...], 0.0)
        oc_ref[...] += jnp.dot(xc, w1, preferred_element_type=jnp.float32)
        of_ref[...] += jnp.dot(xf, w1, preferred_element_type=jnp.float32)


def _loss_kernel(acc_c_ref, acc_f_ref,
                 lic_row_ref, lic_col_ref, liff_row_ref,
                 t1_row_ref, t2_row_ref, t1_col_ref, t2_col_ref, clsf_row_ref,
                 b1_ref, b2_ref, w2_ref, out_ref):
    b1 = b1_ref[...]
    b2 = b2_ref[...]
    w2 = w2_ref[...]
    hc = jnp.maximum(acc_c_ref[...] + b1, 0.0)
    hf = jnp.maximum(acc_f_ref[...] + b1, 0.0)
    aemb = jnp.dot(hc, w2, preferred_element_type=jnp.float32) + b2
    femb = jnp.dot(hf, w2, preferred_element_type=jnp.float32) + b2
    zemb = jnp.dot(jnp.maximum(b1, 0.0), w2,
                   preferred_element_type=jnp.float32) + b2

    lab_ic_row = lic_row_ref[...]
    lab_ic_col = lic_col_ref[...]
    lab_iff_row = liff_row_ref[...]
    t1_row = t1_row_ref[...]
    t2_row = t2_row_ref[...]
    t1_col = t1_col_ref[...]
    t2_col = t2_col_ref[...]
    clsf_row = clsf_row_ref[...]

    cond_col = (t1_col != lab_ic_col) & (t2_col == lab_ic_col)
    cond_row = (t1_row != lab_ic_row) & (t2_row == lab_ic_row)
    condf_row = clsf_row == lab_iff_row
    n_sel = jnp.sum(cond_col.astype(jnp.float32))

    femb2 = femb * femb
    aemb2 = aemb * aemb
    aemb_m = jnp.where(cond_col, aemb, 0.0)
    colnorm = jnp.sqrt(jnp.sum(aemb_m * aemb_m, axis=0, keepdims=True))
    anchor = aemb_m / jnp.maximum(colnorm, 1e-12)
    z2 = zemb * zemb

    posmask = condf_row & (clsf_row == t2_col)
    pm = posmask.astype(jnp.float32)
    lenP = jnp.sum(pm, axis=1, keepdims=True)
    maxP = jnp.max(jnp.where(cond_col, lenP, 0.0))
    norm2P = (jnp.dot(pm, femb2, preferred_element_type=jnp.float32)
              + (maxP - lenP) * z2)
    denP = jnp.maximum(jnp.sqrt(norm2P), 1e-12)
    anchorP = anchor / denP
    num = lax.dot_general(anchorP, femb, (((1,), (1,)), ((), ())),
                          preferred_element_type=jnp.float32) / TEMP
    pad_num = jnp.sum(anchorP * zemb, axis=1, keepdims=True) / TEMP

    maskA = condf_row & (clsf_row == t1_col)
    maskB = cond_row & (t1_row == t2_col)
    ma = maskA.astype(jnp.float32)
    mb = maskB.astype(jnp.float32)
    lenQ = (jnp.sum(ma, axis=1, keepdims=True)
            + jnp.sum(mb, axis=1, keepdims=True))
    maxQ = jnp.max(jnp.where(cond_col, lenQ, 0.0))
    norm2Q = (jnp.dot(ma, femb2, preferred_element_type=jnp.float32)
              + jnp.dot(mb, aemb2, preferred_element_type=jnp.float32)
              + (maxQ - lenQ) * z2)
    denQ = jnp.maximum(jnp.sqrt(norm2Q), 1e-12)
    anchorQ = anchor / denQ
    sF = lax.dot_general(anchorQ, femb, (((1,), (1,)), ((), ())),
                         preferred_element_type=jnp.float32) / TEMP
    sB = lax.dot_general(anchorQ, aemb, (((1,), (1,)), ((), ())),
                         preferred_element_type=jnp.float32) / TEMP
    sFm = jnp.where(maskA, sF, NEG)
    sBm = jnp.where(maskB, sB, NEG)
    has_pad = lenQ < maxQ
    m = jnp.maximum(jnp.max(sFm, axis=1, keepdims=True),
                    jnp.max(sBm, axis=1, keepdims=True))
    m = jnp.maximum(m, jnp.where(has_pad, 0.0, NEG))
    expF = jnp.where(maskA, jnp.exp(sFm - m), 0.0)
    expB = jnp.where(maskB, jnp.exp(sBm - m), 0.0)
    logsum = jnp.log(jnp.sum(expF, axis=1, keepdims=True)
                     + jnp.sum(expB, axis=1, keepdims=True)
                     + (maxQ - lenQ) * jnp.exp(-m))
    numsum = (jnp.sum(jnp.where(posmask, num, 0.0), axis=1, keepdims=True)
              + (maxP - lenP) * pad_num)
    mean_lp = (numsum - maxP * logsum) / maxP
    loss = -(TEMP / BASE_TEMP) * mean_lp
    total = jnp.sum(jnp.where(cond_col, loss, 0.0)) / n_sel
    out_ref[...] = jnp.full((1, 128), total, jnp.float32)


def kernel(label, samples_of_further_pairs, class_of_further_pair,
           idx_further_pair, samples_of_closest_pairs, class_of_closest_pair,
           idx_closest_pair, W1, b1, W2, b2):
    Xc = samples_of_closest_pairs.reshape(128, FLAT)
    Xf = samples_of_further_pairs.reshape(256, FLAT)
    labf = label.astype(jnp.float32)
    idx_all = jnp.concatenate([idx_closest_pair.astype(jnp.int32),
                               idx_further_pair.astype(jnp.int32),
                               jnp.zeros((128,), jnp.int32)])
    ccpf = class_of_closest_pair.astype(jnp.float32)
    t1_row = ccpf[:, 0].reshape(1, 128)
    t2_row = ccpf[:, 1].reshape(1, 128)
    t1_col = ccpf[:, 0].reshape(128, 1)
    t2_col = ccpf[:, 1].reshape(128, 1)
    clsf_row = class_of_further_pair[:, 0].astype(jnp.float32).reshape(1, 256)
    b1r = b1.reshape(1, 128)
    b2r = b2.reshape(1, 128)

    gathered = _sc_gather(labf, idx_all)
    lic_row = gathered[:128].reshape(1, 128)
    lic_col = gathered[:128].reshape(128, 1)
    liff_row = gathered[128:384].reshape(1, 256)

    acc_c, acc_f = pl.pallas_call(
        _mm_kernel,
        grid=(NSTEPS,),
        in_specs=[
            pl.BlockSpec((128, KT), lambda k: (0, k)),
            pl.BlockSpec((256, KT), lambda k: (0, k)),
            pl.BlockSpec((KT, 128), lambda k: (k, 0)),
        ],
        out_specs=[pl.BlockSpec((128, 128), lambda k: (0, 0)),
                   pl.BlockSpec((256, 128), lambda k: (0, 0))],
        out_shape=[jax.ShapeDtypeStruct((128, 128), jnp.float32),
                   jax.ShapeDtypeStruct((256, 128), jnp.float32)],
        compiler_params=pltpu.CompilerParams(
            dimension_semantics=("arbitrary",)),
        interpret=_INTERPRET,
    )(Xc, Xf, W1)

    def full(shape):
        return pl.BlockSpec(shape, lambda: (0,) * len(shape))

    out = pl.pallas_call(
        _loss_kernel,
        in_specs=[
            full((128, 128)), full((256, 128)),
            full((1, 128)), full((128, 1)), full((1, 256)),
            full((1, 128)), full((1, 128)), full((128, 1)), full((128, 1)),
            full((1, 256)), full((1, 128)), full((1, 128)), full((128, 128)),
        ],
        out_specs=full((1, 128)),
        out_shape=jax.ShapeDtypeStruct((1, 128), jnp.float32),
        interpret=_INTERPRET,
    )(acc_c, acc_f, lic_row, lic_col, liff_row,
      t1_row, t2_row, t1_col, t2_col, clsf_row, b1r, b2r, W2)
    return out[0, 0]


# speedup vs baseline: 1.0058x; 1.0058x over previous
import functools

import jax
import jax.numpy as jnp
from jax import lax
from jax.experimental import pallas as pl
from jax.experimental.pallas import tpu as pltpu
from jax.experimental.pallas import tpu_sc as plsc

TEMP = 0.1
BASE_TEMP = 1.0
FLAT = 120000
KT = 3840
NSTEPS = (FLAT + KT - 1) // KT
NEG = -1e30
_INTERPRET = False


def _sc_gather(labf, idx_pad):
    info = plsc.get_sparse_core_info()
    nc = info.num_cores
    mesh = plsc.VectorSubcoreMesh(core_axis_name="c", subcore_axis_name="s")

    @functools.partial(
        pl.kernel, mesh=mesh,
        out_type=jax.ShapeDtypeStruct((512,), jnp.float32),
        scratch_types=[
            pltpu.VMEM((16,), jnp.int32),
            pltpu.VMEM((16,), jnp.float32),
            pltpu.SemaphoreType.DMA,
        ],
    )
    def gather_kernel(lab_hbm, idx_hbm, out_hbm, idxv, outv, sem):
        wid = lax.axis_index("s") * nc + lax.axis_index("c")
        base = wid * 16
        pltpu.sync_copy(idx_hbm.at[pl.ds(base, 16)], idxv)
        pltpu.async_copy(lab_hbm.at[idxv], outv, sem).wait()
        pltpu.sync_copy(outv, out_hbm.at[pl.ds(base, 16)])

    return gather_kernel(labf, idx_pad)


def _loss_kernel(xc_ref, xf_ref, w1_ref,
                 lic_row_ref, lic_col_ref, liff_row_ref,
                 t1_row_ref, t2_row_ref, t1_col_ref, t2_col_ref, clsf_row_ref,
                 b1_ref, b2_ref, w2_ref,
                 out_ref, acc_c, acc_f):
    k = pl.program_id(0)

    @pl.when(k == 0)
    def _init():
        acc_c[...] = jnp.zeros_like(acc_c)
        acc_f[...] = jnp.zeros_like(acc_f)

    @pl.when(k < NSTEPS - 1)
    def _full():
        w1 = w1_ref[...]
        acc_c[...] += jnp.dot(xc_ref[...], w1, preferred_element_type=jnp.float32)
        acc_f[...] += jnp.dot(xf_ref[...], w1, preferred_element_type=jnp.float32)

    @pl.when(k == NSTEPS - 1)
    def _last():
        rem = FLAT - (NSTEPS - 1) * KT
        colmask = lax.broadcasted_iota(jnp.int32, (1, KT), 1) < rem
        rowmask = lax.broadcasted_iota(jnp.int32, (KT, 1), 0) < rem
        xc = jnp.where(colmask, xc_ref[...], 0.0)
        xf = jnp.where(colmask, xf_ref[...], 0.0)
        w1 = jnp.where(rowmask, w1_ref[...], 0.0)
        acc_c[...] += jnp.dot(xc, w1, preferred_element_type=jnp.float32)
        acc_f[...] += jnp.dot(xf, w1, preferred_element_type=jnp.float32)

        b1 = b1_ref[...]
        b2 = b2_ref[...]
        w2 = w2_ref[...]
        hc = jnp.maximum(acc_c[...] + b1, 0.0)
        hf = jnp.maximum(acc_f[...] + b1, 0.0)
        aemb = jnp.dot(hc, w2, preferred_element_type=jnp.float32) + b2
        femb = jnp.dot(hf, w2, preferred_element_type=jnp.float32) + b2
        zemb = jnp.dot(jnp.maximum(b1, 0.0), w2,
                       preferred_element_type=jnp.float32) + b2

        lab_ic_row = lic_row_ref[...]
        lab_ic_col = lic_col_ref[...]
        lab_iff_row = liff_row_ref[...]
        t1_row = t1_row_ref[...]
        t2_row = t2_row_ref[...]
        t1_col = t1_col_ref[...]
        t2_col = t2_col_ref[...]
        clsf_row = clsf_row_ref[...]

        cond_col = (t1_col != lab_ic_col) & (t2_col == lab_ic_col)
        cond_row = (t1_row != lab_ic_row) & (t2_row == lab_ic_row)
        condf_row = clsf_row == lab_iff_row
        n_sel = jnp.sum(cond_col.astype(jnp.float32))

        femb2 = femb * femb
        aemb2 = aemb * aemb
        aemb_m = jnp.where(cond_col, aemb, 0.0)
        colnorm = jnp.sqrt(jnp.sum(aemb_m * aemb_m, axis=0, keepdims=True))
        anchor = aemb_m / jnp.maximum(colnorm, 1e-12)
        z2 = zemb * zemb

        posmask = condf_row & (clsf_row == t2_col)
        pm = posmask.astype(jnp.float32)
        lenP = jnp.sum(pm, axis=1, keepdims=True)
        maxP = jnp.max(jnp.where(cond_col, lenP, 0.0))
        norm2P = (jnp.dot(pm, femb2, preferred_element_type=jnp.float32)
                  + (maxP - lenP) * z2)
        denP = jnp.maximum(jnp.sqrt(norm2P), 1e-12)
        anchorP = anchor / denP
        num = lax.dot_general(anchorP, femb, (((1,), (1,)), ((), ())),
                              preferred_element_type=jnp.float32) / TEMP
        pad_num = jnp.sum(anchorP * zemb, axis=1, keepdims=True) / TEMP

        maskA = condf_row & (clsf_row == t1_col)
        maskB = cond_row & (t1_row == t2_col)
        ma = maskA.astype(jnp.float32)
        mb = maskB.astype(jnp.float32)
        lenQ = (jnp.sum(ma, axis=1, keepdims=True)
                + jnp.sum(mb, axis=1, keepdims=True))
        maxQ = jnp.max(jnp.where(cond_col, lenQ, 0.0))
        norm2Q = (jnp.dot(ma, femb2, preferred_element_type=jnp.float32)
                  + jnp.dot(mb, aemb2, preferred_element_type=jnp.float32)
                  + (maxQ - lenQ) * z2)
        denQ = jnp.maximum(jnp.sqrt(norm2Q), 1e-12)
        anchorQ = anchor / denQ
        sF = lax.dot_general(anchorQ, femb, (((1,), (1,)), ((), ())),
                             preferred_element_type=jnp.float32) / TEMP
        sB = lax.dot_general(anchorQ, aemb, (((1,), (1,)), ((), ())),
                             preferred_element_type=jnp.float32) / TEMP
        sFm = jnp.where(maskA, sF, NEG)
        sBm = jnp.where(maskB, sB, NEG)
        has_pad = lenQ < maxQ
        m = jnp.maximum(jnp.max(sFm, axis=1, keepdims=True),
                        jnp.max(sBm, axis=1, keepdims=True))
        m = jnp.maximum(m, jnp.where(has_pad, 0.0, NEG))
        expF = jnp.where(maskA, jnp.exp(sFm - m), 0.0)
        expB = jnp.where(maskB, jnp.exp(sBm - m), 0.0)
        logsum = jnp.log(jnp.sum(expF, axis=1, keepdims=True)
                         + jnp.sum(expB, axis=1, keepdims=True)
                         + (maxQ - lenQ) * jnp.exp(-m))
        numsum = (jnp.sum(jnp.where(posmask, num, 0.0), axis=1, keepdims=True)
                  + (maxP - lenP) * pad_num)
        mean_lp = (numsum - maxP * logsum) / maxP
        loss = -(TEMP / BASE_TEMP) * mean_lp
        total = jnp.sum(jnp.where(cond_col, loss, 0.0)) / n_sel
        out_ref[...] = jnp.full((1, 128), total, jnp.float32)


def kernel(label, samples_of_further_pairs, class_of_further_pair,
           idx_further_pair, samples_of_closest_pairs, class_of_closest_pair,
           idx_closest_pair, W1, b1, W2, b2):
    Xc = samples_of_closest_pairs.reshape(128, FLAT)
    Xf = samples_of_further_pairs.reshape(256, FLAT)
    labf = label.astype(jnp.float32)
    idx_all = jnp.concatenate([idx_closest_pair.astype(jnp.int32),
                               idx_further_pair.astype(jnp.int32),
                               jnp.zeros((128,), jnp.int32)])
    ccpf = class_of_closest_pair.astype(jnp.float32)
    t1_row = ccpf[:, 0].reshape(1, 128)
    t2_row = ccpf[:, 1].reshape(1, 128)
    t1_col = ccpf[:, 0].reshape(128, 1)
    t2_col = ccpf[:, 1].reshape(128, 1)
    clsf_row = class_of_further_pair[:, 0].astype(jnp.float32).reshape(1, 256)
    b1r = b1.reshape(1, 128)
    b2r = b2.reshape(1, 128)

    gathered = _sc_gather(labf, idx_all)
    lic_row = gathered[:128].reshape(1, 128)
    lic_col = gathered[:128].reshape(128, 1)
    liff_row = gathered[128:384].reshape(1, 256)

    def full(shape):
        return pl.BlockSpec(shape, lambda k: (0,) * len(shape))

    out = pl.pallas_call(
        _loss_kernel,
        grid=(NSTEPS,),
        in_specs=[
            pl.BlockSpec((128, KT), lambda k: (0, k)),
            pl.BlockSpec((256, KT), lambda k: (0, k)),
            pl.BlockSpec((KT, 128), lambda k: (k, 0)),
            full((1, 128)), full((128, 1)), full((1, 256)),
            full((1, 128)), full((1, 128)), full((128, 1)), full((128, 1)),
            full((1, 256)), full((1, 128)), full((1, 128)), full((128, 128)),
        ],
        out_specs=pl.BlockSpec((1, 128), lambda k: (0, 0)),
        out_shape=jax.ShapeDtypeStruct((1, 128), jnp.float32),
        scratch_shapes=[pltpu.VMEM((128, 128), jnp.float32),
                        pltpu.VMEM((256, 128), jnp.float32)],
        compiler_params=pltpu.CompilerParams(
            dimension_semantics=("arbitrary",)),
        interpret=_INTERPRET,
    )(Xc, Xf, W1, lic_row, lic_col, liff_row,
      t1_row, t2_row, t1_col, t2_col, clsf_row, b1r, b2r, W2)
    return out[0, 0]
